# Initial kernel scaffold; baseline (speedup 1.0000x reference)
import jax
import jax.numpy as jnp
from jax.experimental import pallas as pl


def kernel(x, edge_index, W1, b1, W2, b2, ln_g, ln_b, l1_W, l1_b, l2_W, l2_b):
    raise NotImplementedError("write your pallas kernel here")


# speedup vs baseline: 8.1248x; 8.1248x over previous
import functools

import jax
import jax.numpy as jnp
from jax import lax
from jax.experimental import pallas as pl
from jax.experimental.pallas import tpu as pltpu
from jax.experimental.pallas import tpu_sc as plsc

N = 10000
DIN = 128
DH = 256
DOUT = 64
E = 320000

NCORES = 2
NSUB = 16
CHUNK = 128
NG = 160
GRP = 16
NBG = NG // GRP
EPT = NG * CHUNK
E_PAD = NSUB * EPT
ROWS_PAD = 10240
RPT = ROWS_PAD // NSUB
HALF = 128
RBLK = 400
GRID = N // RBLK

_HI = jax.lax.Precision.HIGHEST

_sc_mesh = plsc.VectorSubcoreMesh(core_axis_name="c", subcore_axis_name="s")


@functools.partial(
    pl.kernel,
    out_type=jax.ShapeDtypeStruct((NCORES, ROWS_PAD), jnp.float32),
    mesh=_sc_mesh,
    scratch_types=[
        pltpu.VMEM((NG, CHUNK), jnp.int32),
        pltpu.VMEM((CHUNK,), jnp.float32),
        pltpu.VMEM_SHARED((ROWS_PAD,), jnp.float32),
        pltpu.SemaphoreType.DMA,
    ],
)
def _sc_deg(dstx_hbm, z1_hbm, out_hbm, didx, ones_v, dacc, sem):
    c = lax.axis_index("c")
    s = lax.axis_index("s")
    r0 = s * RPT
    for i in range(CHUNK // 16):
        ones_v[pl.ds(i * 16, 16)] = jnp.ones((16,), jnp.float32)
    pltpu.sync_copy(z1_hbm.at[pl.ds(r0, RPT)], dacc.at[pl.ds(r0, RPT)])
    pltpu.sync_copy(dstx_hbm.at[s], didx)
    plsc.subcore_barrier()

    def body(g, carry):
        @pl.when((g % NCORES) == c)
        def _():
            pltpu.sync_copy(ones_v, dacc.at[didx.at[g]], add=True)
        return carry

    lax.fori_loop(0, NG, body, 0)
    plsc.subcore_barrier()
    pltpu.sync_copy(dacc.at[pl.ds(r0, RPT)], out_hbm.at[c, pl.ds(r0, RPT)])


@functools.partial(
    pl.kernel,
    out_type=jax.ShapeDtypeStruct((ROWS_PAD, DH), jnp.float32),
    mesh=_sc_mesh,
    scratch_types=[
        pltpu.VMEM((GRP, CHUNK), jnp.int32),
        pltpu.VMEM((GRP, CHUNK), jnp.int32),
        pltpu.VMEM((CHUNK, HALF), jnp.float32),
        pltpu.VMEM((CHUNK, HALF), jnp.float32),
        pltpu.VMEM_SHARED((ROWS_PAD, HALF), jnp.float32),
        pltpu.SemaphoreType.DMA,
        pltpu.SemaphoreType.DMA,
    ],
)
def _sc_scatter(hsv_hbm, srcx_hbm, dstx_hbm, z2_hbm, out_hbm,
                sidx, didx, buf_a, buf_b, acc, sem_a, sem_b):
    c = lax.axis_index("c")
    s = lax.axis_index("s")
    r0 = s * RPT
    pltpu.sync_copy(z2_hbm.at[pl.ds(r0, RPT)], acc.at[pl.ds(r0, RPT)])
    plsc.subcore_barrier()

    def group(nb, carry):
        pltpu.sync_copy(srcx_hbm.at[c, s, pl.ds(nb * GRP, GRP)], sidx)
        pltpu.sync_copy(dstx_hbm.at[s, pl.ds(nb * GRP, GRP)], didx)
        pltpu.async_copy(hsv_hbm.at[sidx.at[0]], buf_a, sem_a)

        def body(g, carry2):
            @pl.when((g % 2) == 0)
            def _():
                pltpu.make_async_copy(hsv_hbm.at[sidx.at[g]], buf_a, sem_a).wait()
                @pl.when(g + 1 < GRP)
                def _():
                    pltpu.async_copy(hsv_hbm.at[sidx.at[g + 1]], buf_b, sem_b)
                pltpu.sync_copy(buf_a, acc.at[didx.at[g]], add=True)

            @pl.when((g % 2) == 1)
            def _():
                pltpu.make_async_copy(hsv_hbm.at[sidx.at[g]], buf_b, sem_b).wait()
                @pl.when(g + 1 < GRP)
                def _():
                    pltpu.async_copy(hsv_hbm.at[sidx.at[g + 1]], buf_a, sem_a)
                pltpu.sync_copy(buf_b, acc.at[didx.at[g]], add=True)

            return carry2

        lax.fori_loop(0, GRP, body, 0)
        return carry

    lax.fori_loop(0, NBG, group, 0)
    plsc.subcore_barrier()
    pltpu.sync_copy(acc.at[pl.ds(r0, RPT)],
                    out_hbm.at[pl.ds(r0, RPT), pl.ds(c * HALF, HALF)])


def _dinv_of(deg_ref):
    d = jnp.sum(deg_ref[...], axis=1) + 1.0
    return lax.rsqrt(d)[:, None]


def _tc1_body(x_ref, w1_ref, deg_ref, o_ref):
    h = jnp.dot(x_ref[...], w1_ref[...], precision=_HI)
    o_ref[...] = h * _dinv_of(deg_ref)


def _tc2_body(raw_ref, hs_ref, deg_ref, b1_ref, g_ref, bln_ref, w2_ref, o_ref):
    dinv = _dinv_of(deg_ref)
    t = (raw_ref[...] + hs_ref[...]) * dinv + b1_ref[...]
    t = jnp.maximum(t, 0.0)
    mu = jnp.mean(t, axis=1, keepdims=True)
    var = jnp.mean((t - mu) ** 2, axis=1, keepdims=True)
    z = (t - mu) * lax.rsqrt(var + 1e-5) * g_ref[...] + bln_ref[...]
    h2 = jnp.dot(z, w2_ref[...], precision=_HI)
    o_ref[...] = h2 * dinv


def _tc3_body(raw_ref, hs_ref, deg_ref, b2_ref, w1_ref, lb1_ref,
              w2_ref, lb2_ref, emb_ref, lp_ref):
    dinv = _dinv_of(deg_ref)
    emb = (raw_ref[...] + hs_ref[...]) * dinv + b2_ref[...]
    emb_ref[...] = emb
    r = jnp.maximum(emb, 0.0)
    t = jnp.dot(r, w1_ref[...], precision=_HI) + lb1_ref[...]
    u = jnp.dot(t, w2_ref[...], precision=_HI) + lb2_ref[...]
    m = jnp.max(u, axis=1, keepdims=True)
    lse = jnp.log(jnp.sum(jnp.exp(u - m), axis=1, keepdims=True)) + m
    lp_ref[...] = u - lse


def _row_spec(cols):
    return pl.BlockSpec((RBLK, cols), lambda i: (i, 0))


def _full_spec(r, c):
    return pl.BlockSpec((r, c), lambda i: (0, 0))


_deg_spec = pl.BlockSpec((RBLK, NCORES), lambda i: (i, 0))

_tc1 = pl.pallas_call(
    _tc1_body,
    grid=(GRID,),
    in_specs=[_row_spec(DIN), _full_spec(DIN, DH), _deg_spec],
    out_specs=_row_spec(DH),
    out_shape=jax.ShapeDtypeStruct((N, DH), jnp.float32),
)

_tc2 = pl.pallas_call(
    _tc2_body,
    grid=(GRID,),
    in_specs=[_row_spec(DH), _row_spec(DH), _deg_spec, _full_spec(1, DH),
              _full_spec(1, DH), _full_spec(1, DH), _full_spec(DH, DH)],
    out_specs=_row_spec(DH),
    out_shape=jax.ShapeDtypeStruct((N, DH), jnp.float32),
)

_tc3 = pl.pallas_call(
    _tc3_body,
    grid=(GRID,),
    in_specs=[_row_spec(DH), _row_spec(DH), _deg_spec, _full_spec(1, DH),
              _full_spec(DH, DH), _full_spec(1, DH), _full_spec(DH, DOUT),
              _full_spec(1, DOUT)],
    out_specs=[_row_spec(DH), _row_spec(DOUT)],
    out_shape=[jax.ShapeDtypeStruct((N, DH), jnp.float32),
               jax.ShapeDtypeStruct((N, DOUT), jnp.float32)],
)


def kernel(x, edge_index, W1, b1, W2, b2, ln_g, ln_b, l1_W, l1_b, l2_W, l2_b):
    src = edge_index[0].astype(jnp.int32)
    dst = edge_index[1].astype(jnp.int32)
    pad = E_PAD - E
    srcp = jnp.concatenate([src, jnp.zeros((pad,), jnp.int32)])
    dstp = jnp.concatenate([dst, jnp.full((pad,), N, jnp.int32)])
    srcx = jnp.stack([srcp * 2, srcp * 2 + 1]).reshape(NCORES, NSUB, NG, CHUNK)
    dstx = dstp.reshape(NSUB, NG, CHUNK)
    z2 = jnp.zeros((ROWS_PAD, HALF), jnp.float32)
    z1 = jnp.zeros((ROWS_PAD,), jnp.float32)

    b1r = b1.reshape(1, DH)
    b2r = b2.reshape(1, DH)
    gr = ln_g.reshape(1, DH)
    br = ln_b.reshape(1, DH)
    l1br = l1_b.reshape(1, DH)
    l2br = l2_b.reshape(1, DOUT)

    degp = _sc_deg(dstx, z1).T
    hs1 = _tc1(x, W1, degp)
    raw1 = _sc_scatter(hs1.reshape(2 * N, HALF), srcx, dstx, z2)
    hs2 = _tc2(raw1, hs1, degp, b1r, gr, br, W2)
    raw2 = _sc_scatter(hs2.reshape(2 * N, HALF), srcx, dstx, z2)
    emb, logp = _tc3(raw2, hs2, degp, b2r, l1_W, l1br, l2_W, l2br)
    return emb, logp
